# baseline (speedup 1.0000x reference)
import functools

import jax
import jax.numpy as jnp
from jax import lax
from jax.experimental import pallas as pl
from jax.experimental.pallas import tpu as pltpu
from jax.experimental.pallas import tpu_sc as plsc

N_NODES = 10000
N_EDGES = 320000
D_FEAT = 128
D_EDGE = 16
N_GRAPHS = 64
HID = 10
HP = 16

NC, NS = 2, 16
NW = NC * NS
E_HALF = N_EDGES // 2
E_PER_W = E_HALF // NW
GRP = 125
EPR = 125
GPC = 8
CHUNK = 1000
N_CHUNKS = E_PER_W // CHUNK
N_PAD = 10240
ROWS_PER_TILE = N_PAD // NS
ZB = 128


def _node_proj_body(x_ref, w_ref, b_ref, o_ref):
    o_ref[...] = (
        jnp.dot(x_ref[...], w_ref[...], preferred_element_type=jnp.float32)
        + b_ref[...]
    )


def _node_proj(x8, w_stack, b_t):
    return pl.pallas_call(
        _node_proj_body,
        grid=(2,),
        in_specs=[
            pl.BlockSpec((N_PAD // 16, 8 * D_FEAT), lambda i: (i, 0)),
            pl.BlockSpec((8 * D_FEAT, 128), lambda i: (0, 0)),
            pl.BlockSpec((1, 128), lambda i: (0, 0)),
        ],
        out_specs=pl.BlockSpec((N_PAD // 16, 128), lambda i: (i, 0)),
        out_shape=jax.ShapeDtypeStruct((N_PAD // 8, 128), jnp.float32),
    )(x8, w_stack, b_t)


def _edge_proj_body(x_ref, w_ref, o_ref):
    parts = [
        jnp.dot(x_ref[:, a, :], w_ref[...], preferred_element_type=jnp.float32)
        for a in range(8)
    ]
    o_ref[...] = jnp.concatenate(parts, axis=1)


def _edge_proj(ea3, w_e, half):
    return pl.pallas_call(
        _edge_proj_body,
        grid=(20,),
        in_specs=[
            pl.BlockSpec((1000, 8, D_EDGE), lambda i: (i + half * 20, 0, 0)),
            pl.BlockSpec((D_EDGE, HP), lambda i: (0, 0)),
        ],
        out_specs=pl.BlockSpec((1000, 128), lambda i: (i, 0)),
        out_shape=jax.ShapeDtypeStruct((N_EDGES // 16, 128), jnp.float32),
    )(ea3, w_e)


def _mp_body(np_hbm, ep_hbm, ei_hbm, out_hbm, src_i, dst_i, rows_v, ep_v, zbuf,
             acc, gsem, esem):
    cid = lax.axis_index("c")
    sid = lax.axis_index("s")
    wid = sid * NC + cid

    def _z(i, c):
        zbuf[i] = jnp.zeros((HP,), jnp.float32)
        return c

    lax.fori_loop(0, ZB, _z, 0)
    for t in range(ROWS_PER_TILE // ZB):
        pltpu.sync_copy(zbuf, acc.at[pl.ds(sid * ROWS_PER_TILE + t * ZB, ZB)])
    plsc.subcore_barrier()

    nrows = N_CHUNKS * GPC
    pltpu.sync_copy(ei_hbm.at[0, pl.ds(wid * nrows, nrows)], src_i)
    pltpu.sync_copy(ei_hbm.at[1, pl.ds(wid * nrows, nrows)], dst_i)

    ep_cp = [None, None]
    g_cps = [None, None]

    def _start(c):
        p = c & 1
        m = wid * N_CHUNKS + c
        ep_cp[p] = pltpu.async_copy(
            ep_hbm.at[pl.ds(m * EPR, EPR)], ep_v.at[p], esem
        )
        g_cps[p] = [
            pltpu.async_copy(
                np_hbm.at[src_i.at[c * GPC + j]],
                rows_v.at[p, pl.ds(j * GRP, GRP)],
                gsem,
            )
            for j in range(GPC)
        ]

    _start(0)
    for c in range(N_CHUNKS):
        p = c & 1
        if c + 1 < N_CHUNKS:
            _start(c + 1)
        ep_cp[p].wait()
        for cp in g_cps[p]:
            cp.wait()

        def _relu_add(r, carry):
            for k in range(8):
                e = 8 * r + k
                rows_v[p, e] = jnp.maximum(
                    rows_v[p, e] + ep_v[p, r, pl.ds(k * HP, HP)], 0.0
                )
            return carry

        lax.fori_loop(0, EPR, _relu_add, 0, unroll=2)
        for j in range(GPC):
            pltpu.sync_copy(
                rows_v.at[p, pl.ds(j * GRP, GRP)],
                acc.at[dst_i.at[c * GPC + j]],
                add=True,
            )

    plsc.subcore_barrier()
    pltpu.sync_copy(
        acc.at[pl.ds(sid * ROWS_PER_TILE, ROWS_PER_TILE)],
        out_hbm.at[cid, pl.ds(sid * ROWS_PER_TILE, ROWS_PER_TILE)],
    )


@functools.cache
def _make_mp_call():
    return pl.kernel(
        _mp_body,
        out_type=jax.ShapeDtypeStruct((NC, N_PAD, HP), jnp.float32),
        mesh=plsc.VectorSubcoreMesh(
            core_axis_name="c", subcore_axis_name="s", num_cores=NC, num_subcores=NS
        ),
        compiler_params=pltpu.CompilerParams(use_tc_tiling_on_sc=False),
        scratch_types=[
            pltpu.VMEM((N_CHUNKS * GPC, GRP), jnp.int32),
            pltpu.VMEM((N_CHUNKS * GPC, GRP), jnp.int32),
            pltpu.VMEM((2, CHUNK, HP), jnp.float32),
            pltpu.VMEM((2, EPR, 128), jnp.float32),
            pltpu.VMEM((ZB, HP), jnp.float32),
            pltpu.VMEM_SHARED((N_PAD, HP), jnp.float32),
            pltpu.SemaphoreType.DMA,
            pltpu.SemaphoreType.DMA,
        ],
    )


def _tail_body(xp_ref, xq_ref, b_ref, w1_ref, b1_ref, w2_ref, b2_ref, w3_ref,
               b3_ref, o_ref, acc_ref):
    i = pl.program_id(0)
    xc = (xp_ref[0] + xp_ref[1]) + (xq_ref[0] + xq_ref[1])
    x2 = jnp.maximum(
        jnp.dot(xc, w1_ref[...], preferred_element_type=jnp.float32) + b1_ref[...],
        0.0,
    )
    bids = b_ref[0]
    gids = lax.broadcasted_iota(jnp.int32, (N_GRAPHS, 1024), 0)
    onehot = (bids == gids).astype(jnp.float32)
    part = jnp.dot(onehot, x2, preferred_element_type=jnp.float32, precision=lax.Precision.HIGHEST)

    @pl.when(i == 0)
    def _():
        acc_ref[...] = part

    @pl.when(i > 0)
    def _():
        acc_ref[...] += part

    @pl.when(i == pl.num_programs(0) - 1)
    def _():
        g = acc_ref[...]
        g2 = jnp.maximum(
            jnp.dot(g, w2_ref[...], preferred_element_type=jnp.float32) + b2_ref[...],
            0.0,
        )
        o_ref[...] = (
            jnp.dot(g2, w3_ref[...], preferred_element_type=jnp.float32) + b3_ref[...]
        )


def _tail(x_a, x_b, batch3, w1p, b1p, w2p, b2p, w3p, b3p):
    return pl.pallas_call(
        _tail_body,
        grid=(10,),
        in_specs=[
            pl.BlockSpec((NC, 1024, HP), lambda i: (0, i, 0)),
            pl.BlockSpec((NC, 1024, HP), lambda i: (0, i, 0)),
            pl.BlockSpec((1, 1, 1024), lambda i: (i, 0, 0)),
            pl.BlockSpec((HP, 128), lambda i: (0, 0)),
            pl.BlockSpec((1, 128), lambda i: (0, 0)),
            pl.BlockSpec((128, 128), lambda i: (0, 0)),
            pl.BlockSpec((1, 128), lambda i: (0, 0)),
            pl.BlockSpec((128, 128), lambda i: (0, 0)),
            pl.BlockSpec((1, 128), lambda i: (0, 0)),
        ],
        out_specs=pl.BlockSpec((N_GRAPHS, 128), lambda i: (0, 0)),
        out_shape=jax.ShapeDtypeStruct((N_GRAPHS, 128), jnp.float32),
        scratch_shapes=[pltpu.VMEM((N_GRAPHS, 128), jnp.float32)],
    )(x_a, x_b, batch3, w1p, b1p, w2p, b2p, w3p, b3p)


def kernel(edge_index, node_attr, edge_attr, batch, W_msg, b_msg, W1, b1, W2, b2,
           W3, b3):
    f32 = jnp.float32
    w_np = jnp.pad(W_msg[:D_FEAT], ((0, 0), (0, HP - HID)))
    w_stack = jnp.kron(jnp.eye(8, dtype=f32), w_np)
    b_t = jnp.tile(jnp.pad(b_msg, (0, HP - HID)), 8).reshape(1, 128)
    w_e = jnp.pad(W_msg[D_FEAT:], ((0, 0), (0, HP - HID)))
    w1p = jnp.zeros((HP, 128), f32).at[:HID, :5].set(W1)
    b1p = jnp.zeros((1, 128), f32).at[0, :5].set(b1)
    w2p = jnp.zeros((128, 128), f32).at[:5, :5].set(W2)
    b2p = jnp.zeros((1, 128), f32).at[0, :5].set(b2)
    w3p = jnp.zeros((128, 128), f32).at[:5, :1].set(W3)
    b3p = jnp.zeros((1, 128), f32).at[0, :1].set(b3)

    na_p = jnp.pad(node_attr, ((0, N_PAD - N_NODES), (0, 0)))
    np_pk = _node_proj(na_p.reshape(N_PAD // 8, 8 * D_FEAT), w_stack, b_t)
    node_proj = np_pk.reshape(N_PAD, HP)

    ea3 = edge_attr.reshape(N_EDGES // 8, 8, D_EDGE)
    ei3 = edge_index.astype(jnp.int32).reshape(2, N_EDGES // GRP, GRP)
    rows_h = E_HALF // GRP
    mp = _make_mp_call()
    ep_a = _edge_proj(ea3, w_e, 0)
    x_a = mp(node_proj, ep_a, ei3[:, :rows_h])
    ep_b = _edge_proj(ea3, w_e, 1)
    x_b = mp(node_proj, ep_b, ei3[:, rows_h:])

    batch3 = jnp.pad(
        batch.astype(jnp.int32), (0, N_PAD - N_NODES), constant_values=N_GRAPHS
    ).reshape(10, 1, 1024)
    out = _tail(x_a, x_b, batch3, w1p, b1p, w2p, b2p, w3p, b3p)
    return out[:, :1]


# speedup vs baseline: 1.0173x; 1.0173x over previous
import functools

import jax
import jax.numpy as jnp
from jax import lax
from jax.experimental import pallas as pl
from jax.experimental.pallas import tpu as pltpu
from jax.experimental.pallas import tpu_sc as plsc

N_NODES = 10000
N_EDGES = 320000
D_FEAT = 128
D_EDGE = 16
N_GRAPHS = 64
HID = 10
HP = 16

NC, NS = 2, 16
NW = NC * NS
E_HALF = N_EDGES // 2
E_PER_W = E_HALF // NW
GRP = 125
EPR = 125
GPC = 8
CHUNK = 1000
N_CHUNKS = E_PER_W // CHUNK
N_PAD = 10240
ROWS_PER_TILE = N_PAD // NS
ZB = 128


def _node_proj_body(x_ref, w_ref, b_ref, o_ref):
    o_ref[...] = (
        jnp.dot(x_ref[...], w_ref[...], preferred_element_type=jnp.float32)
        + b_ref[...]
    )


def _node_proj(x8, w_stack, b_t):
    return pl.pallas_call(
        _node_proj_body,
        grid=(2,),
        in_specs=[
            pl.BlockSpec((N_PAD // 16, 8 * D_FEAT), lambda i: (i, 0)),
            pl.BlockSpec((8 * D_FEAT, 128), lambda i: (0, 0)),
            pl.BlockSpec((1, 128), lambda i: (0, 0)),
        ],
        out_specs=pl.BlockSpec((N_PAD // 16, 128), lambda i: (i, 0)),
        out_shape=jax.ShapeDtypeStruct((N_PAD // 8, 128), jnp.float32),
    )(x8, w_stack, b_t)


def _edge_proj_body(x_ref, w_ref, o_ref):
    parts = [
        jnp.dot(x_ref[:, a, :], w_ref[...], preferred_element_type=jnp.float32)
        for a in range(8)
    ]
    o_ref[...] = jnp.concatenate(parts, axis=1)


def _edge_proj(ea3, w_e, half):
    return pl.pallas_call(
        _edge_proj_body,
        grid=(20,),
        in_specs=[
            pl.BlockSpec((1000, 8, D_EDGE), lambda i: (i + half * 20, 0, 0)),
            pl.BlockSpec((D_EDGE, HP), lambda i: (0, 0)),
        ],
        out_specs=pl.BlockSpec((1000, 128), lambda i: (i, 0)),
        out_shape=jax.ShapeDtypeStruct((N_EDGES // 16, 128), jnp.float32),
    )(ea3, w_e)


def _mp_body(np_hbm, ep_hbm, ei_hbm, out_hbm, src_i, dst_i, rows_v, ep_v, zbuf,
             acc, gsem):
    cid = lax.axis_index("c")
    sid = lax.axis_index("s")
    wid = sid * NC + cid

    def _z(i, c):
        zbuf[i] = jnp.zeros((HP,), jnp.float32)
        return c

    lax.fori_loop(0, ZB, _z, 0)
    for t in range(ROWS_PER_TILE // ZB):
        pltpu.sync_copy(zbuf, acc.at[pl.ds(sid * ROWS_PER_TILE + t * ZB, ZB)])
    plsc.subcore_barrier()

    for c in range(N_CHUNKS):
        m = wid * N_CHUNKS + c
        rbase = m * GPC
        pltpu.sync_copy(ei_hbm.at[0, pl.ds(rbase, GPC)], src_i)
        pltpu.sync_copy(ei_hbm.at[1, pl.ds(rbase, GPC)], dst_i)
        pltpu.sync_copy(ep_hbm.at[pl.ds(m * EPR, EPR)], ep_v)
        cps = [
            pltpu.async_copy(
                np_hbm.at[src_i.at[j]], rows_v.at[pl.ds(j * GRP, GRP)], gsem
            )
            for j in range(GPC)
        ]
        for cp in cps:
            cp.wait()

        def _relu_add(r, carry):
            for k in range(8):
                e = 8 * r + k
                rows_v[e] = jnp.maximum(
                    rows_v[e] + ep_v[r, pl.ds(k * HP, HP)], 0.0
                )
            return carry

        lax.fori_loop(0, EPR, _relu_add, 0)
        for j in range(GPC):
            pltpu.sync_copy(
                rows_v.at[pl.ds(j * GRP, GRP)], acc.at[dst_i.at[j]], add=True
            )

    plsc.subcore_barrier()
    pltpu.sync_copy(
        acc.at[pl.ds(sid * ROWS_PER_TILE, ROWS_PER_TILE)],
        out_hbm.at[cid, pl.ds(sid * ROWS_PER_TILE, ROWS_PER_TILE)],
    )


@functools.cache
def _make_mp_call():
    return pl.kernel(
        _mp_body,
        out_type=jax.ShapeDtypeStruct((NC, N_PAD, HP), jnp.float32),
        mesh=plsc.VectorSubcoreMesh(
            core_axis_name="c", subcore_axis_name="s", num_cores=NC, num_subcores=NS
        ),
        compiler_params=pltpu.CompilerParams(use_tc_tiling_on_sc=False),
        scratch_types=[
            pltpu.VMEM((GPC, GRP), jnp.int32),
            pltpu.VMEM((GPC, GRP), jnp.int32),
            pltpu.VMEM((CHUNK, HP), jnp.float32),
            pltpu.VMEM((EPR, 128), jnp.float32),
            pltpu.VMEM((ZB, HP), jnp.float32),
            pltpu.VMEM_SHARED((N_PAD, HP), jnp.float32),
            pltpu.SemaphoreType.DMA,
        ],
    )


def _tail_body(xp_ref, xq_ref, b_ref, w1_ref, b1_ref, w2_ref, b2_ref, w3_ref,
               b3_ref, o_ref, acc_ref):
    i = pl.program_id(0)
    xc = (xp_ref[0] + xp_ref[1]) + (xq_ref[0] + xq_ref[1])
    x2 = jnp.maximum(
        jnp.dot(xc, w1_ref[...], preferred_element_type=jnp.float32) + b1_ref[...],
        0.0,
    )
    bids = b_ref[0]
    gids = lax.broadcasted_iota(jnp.int32, (N_GRAPHS, 1024), 0)
    onehot = (bids == gids).astype(jnp.float32)
    part = jnp.dot(onehot, x2, preferred_element_type=jnp.float32, precision=lax.Precision.HIGHEST)

    @pl.when(i == 0)
    def _():
        acc_ref[...] = part

    @pl.when(i > 0)
    def _():
        acc_ref[...] += part

    @pl.when(i == pl.num_programs(0) - 1)
    def _():
        g = acc_ref[...]
        g2 = jnp.maximum(
            jnp.dot(g, w2_ref[...], preferred_element_type=jnp.float32) + b2_ref[...],
            0.0,
        )
        o_ref[...] = (
            jnp.dot(g2, w3_ref[...], preferred_element_type=jnp.float32) + b3_ref[...]
        )


def _tail(x_a, x_b, batch3, w1p, b1p, w2p, b2p, w3p, b3p):
    return pl.pallas_call(
        _tail_body,
        grid=(10,),
        in_specs=[
            pl.BlockSpec((NC, 1024, HP), lambda i: (0, i, 0)),
            pl.BlockSpec((NC, 1024, HP), lambda i: (0, i, 0)),
            pl.BlockSpec((1, 1, 1024), lambda i: (i, 0, 0)),
            pl.BlockSpec((HP, 128), lambda i: (0, 0)),
            pl.BlockSpec((1, 128), lambda i: (0, 0)),
            pl.BlockSpec((128, 128), lambda i: (0, 0)),
            pl.BlockSpec((1, 128), lambda i: (0, 0)),
            pl.BlockSpec((128, 128), lambda i: (0, 0)),
            pl.BlockSpec((1, 128), lambda i: (0, 0)),
        ],
        out_specs=pl.BlockSpec((N_GRAPHS, 128), lambda i: (0, 0)),
        out_shape=jax.ShapeDtypeStruct((N_GRAPHS, 128), jnp.float32),
        scratch_shapes=[pltpu.VMEM((N_GRAPHS, 128), jnp.float32)],
    )(x_a, x_b, batch3, w1p, b1p, w2p, b2p, w3p, b3p)


def kernel(edge_index, node_attr, edge_attr, batch, W_msg, b_msg, W1, b1, W2, b2,
           W3, b3):
    f32 = jnp.float32
    w_np = jnp.pad(W_msg[:D_FEAT], ((0, 0), (0, HP - HID)))
    w_stack = jnp.kron(jnp.eye(8, dtype=f32), w_np)
    b_t = jnp.tile(jnp.pad(b_msg, (0, HP - HID)), 8).reshape(1, 128)
    w_e = jnp.pad(W_msg[D_FEAT:], ((0, 0), (0, HP - HID)))
    w1p = jnp.zeros((HP, 128), f32).at[:HID, :5].set(W1)
    b1p = jnp.zeros((1, 128), f32).at[0, :5].set(b1)
    w2p = jnp.zeros((128, 128), f32).at[:5, :5].set(W2)
    b2p = jnp.zeros((1, 128), f32).at[0, :5].set(b2)
    w3p = jnp.zeros((128, 128), f32).at[:5, :1].set(W3)
    b3p = jnp.zeros((1, 128), f32).at[0, :1].set(b3)

    na_p = jnp.pad(node_attr, ((0, N_PAD - N_NODES), (0, 0)))
    np_pk = _node_proj(na_p.reshape(N_PAD // 8, 8 * D_FEAT), w_stack, b_t)
    node_proj = np_pk.reshape(N_PAD, HP)

    ea3 = edge_attr.reshape(N_EDGES // 8, 8, D_EDGE)
    ei3 = edge_index.astype(jnp.int32).reshape(2, N_EDGES // GRP, GRP)
    rows_h = E_HALF // GRP
    mp = _make_mp_call()
    ep_a = _edge_proj(ea3, w_e, 0)
    x_a = mp(node_proj, ep_a, ei3[:, :rows_h])
    ep_b = _edge_proj(ea3, w_e, 1)
    x_b = mp(node_proj, ep_b, ei3[:, rows_h:])

    batch3 = jnp.pad(
        batch.astype(jnp.int32), (0, N_PAD - N_NODES), constant_values=N_GRAPHS
    ).reshape(10, 1, 1024)
    out = _tail(x_a, x_b, batch3, w1p, b1p, w2p, b2p, w3p, b3p)
    return out[:, :1]
